# baseline (speedup 1.0000x reference)
import jax
import jax.numpy as jnp
from jax import lax
from jax.experimental import pallas as pl
from jax.experimental.pallas import tpu as pltpu
from jax.experimental.pallas import tpu_sc as plsc

_B, _N, _P, _R = 4, 48, 17, 8
_NQ = 4
_NI = _N // _NQ
_NRQ = 2
_RL = _R // _NRQ
_TW = _NI * _N * _P
_MW = _NI * _N * _R
_OW = _RL * _NI * _N


def _sc_body(trans_hbm, tmask_hbm, rules_hbm, w_hbm, b_hbm, out_hbm,
             trans_v, tm_v, rules_v, w_v, b_v, out_v, sem1, sem2):
    wid = lax.axis_index("s") * 2 + lax.axis_index("c")
    b = wid // 8
    rq = (wid % 8) // _NQ
    q = wid % _NQ

    c1 = pltpu.make_async_copy(trans_hbm.at[b * _NQ + q], trans_v, sem1)
    c1.start()
    c2 = pltpu.make_async_copy(tmask_hbm.at[b * _NQ + q], tm_v, sem2)
    c2.start()
    pltpu.sync_copy(rules_hbm, rules_v)
    pltpu.sync_copy(w_hbm, w_v)
    pltpu.sync_copy(b_hbm, b_v)
    c1.wait()
    c2.wait()

    lane = lax.iota(jnp.int32, 16)
    lane17 = lane * 17
    lane8 = lane * 8

    def bcast_i(ref, k):
        return plsc.load_gather(ref, [jnp.full((16,), 0, jnp.int32) + k])

    for rl in range(_RL):
        rel = _RL * rq + rl
        pre0 = [lane17 + bcast_i(rules_v, 12 * rel + 2 * m) for m in range(6)]
        pre1 = [lane17 + bcast_i(rules_v, 12 * rel + 2 * m + 1) for m in range(6)]
        w = [bcast_i(w_v, 6 * rel + k) for k in range(6)]
        bias0 = bcast_i(b_v, 2 * rel)
        bias1 = bcast_i(b_v, 2 * rel + 1)

        def body(i, carry):
            base_i = i * (_N * _P)
            tm_base = i * (_N * _R) + rel

            rms = []
            for m in range(6):
                v = plsc.load_gather(trans_v, [pre0[m] + base_i])
                for jb in range(1, 3):
                    v = jnp.maximum(v, plsc.load_gather(
                        trans_v, [pre0[m] + (base_i + jb * 272)]))
                rms.append(jnp.max(v))

            for jb in range(3):
                off = base_i + jb * 272
                acc0 = bias0
                acc1 = bias1
                for m in range(3):
                    t1v = plsc.load_gather(trans_v, [pre1[m] + off])
                    acc0 = acc0 + w[m] * jnp.exp(t1v + rms[m])
                for m in range(3, 6):
                    t1v = plsc.load_gather(trans_v, [pre1[m] + off])
                    acc1 = acc1 + w[m] * jnp.exp(t1v + rms[m])
                tmv = plsc.load_gather(tm_v, [lane8 + (tm_base + jb * 128)])
                res = jnp.where(tmv == 0, acc0, acc1)
                out_v[pl.ds(rl * (_NI * _N) + i * _N + jb * 16, 16)] = res
            return carry

        lax.fori_loop(0, _NI, body, 0)

    pltpu.sync_copy(out_v, out_hbm.at[wid])


def _sc_call(trans_rows, tmask_rows, rules_flat, w_flat, biases):
    mesh = plsc.VectorSubcoreMesh(core_axis_name="c", subcore_axis_name="s")
    f = pl.kernel(
        _sc_body,
        out_type=jax.ShapeDtypeStruct((32, _OW), jnp.float32),
        mesh=mesh,
        compiler_params=pltpu.CompilerParams(needs_layout_passes=False),
        scratch_types=[
            pltpu.VMEM((_TW,), jnp.float32),
            pltpu.VMEM((_MW,), jnp.int32),
            pltpu.VMEM((96,), jnp.int32),
            pltpu.VMEM((48,), jnp.float32),
            pltpu.VMEM((16,), jnp.float32),
            pltpu.VMEM((_OW,), jnp.float32),
            pltpu.SemaphoreType.DMA,
            pltpu.SemaphoreType.DMA,
        ],
    )
    return f(trans_rows, tmask_rows, rules_flat, w_flat, biases)


def kernel(transitions, type_mask, rules, weights, biases, t_sections, c_sections):
    B, N, _, P = transitions.shape
    R = type_mask.shape[-1]
    trans_rows = transitions.reshape(B * _NQ, _TW)
    tmask_rows = type_mask.reshape(B * _NQ, _MW)
    rules_flat = rules.reshape(96).astype(jnp.int32)
    w_flat = weights.reshape(48)
    out = _sc_call(trans_rows, tmask_rows, rules_flat, w_flat, biases)
    out = out.reshape(B, _NRQ, _NQ, _RL, _NI, N)
    out = out.transpose(0, 2, 4, 5, 1, 3)
    return out.reshape(B, N, N, R)


# speedup vs baseline: 1.3077x; 1.3077x over previous
import jax
import jax.numpy as jnp
from jax.experimental import pallas as pl
from jax.experimental.pallas import tpu as pltpu

_HIGH = jax.lax.Precision.HIGHEST


def _tc_body(trans_ref, tmask_ref, rules_ref, weights_ref, biases_ref,
             out_ref):
    _, N, _, P = trans_ref.shape
    R = tmask_ref.shape[-1]
    Nc = rules_ref.shape[0]

    trans = trans_ref[0]
    th = trans.reshape(N * N, P)

    pgrid = jax.lax.broadcasted_iota(jnp.int32, (Nc, P), 1)
    oh0 = (pgrid == rules_ref[:, 0:1]).astype(jnp.float32)
    oh1 = (pgrid == rules_ref[:, 1:2]).astype(jnp.float32)
    dn = (((1,), (1,)), ((), ()))

    rm = jnp.max(trans, axis=1)
    rmg = jax.lax.dot_general(rm, oh0, dn,
                              preferred_element_type=jnp.float32,
                              precision=_HIGH)
    t1 = jax.lax.dot_general(th, oh1, dn,
                             preferred_element_type=jnp.float32,
                             precision=_HIGH).reshape(N, N, Nc)
    scores = jnp.exp(rmg[:, None, :] + t1).reshape(N * N, Nc)

    cgrid = jax.lax.broadcasted_iota(jnp.int32, (Nc, R), 0)
    rgrid = jax.lax.broadcasted_iota(jnp.int32, (Nc, R), 1)
    rgrid1 = jax.lax.broadcasted_iota(jnp.int32, (1, R), 1)
    W0 = jnp.zeros((Nc, R), jnp.float32)
    W1 = jnp.zeros((Nc, R), jnp.float32)
    b0 = jnp.zeros((1, R), jnp.float32)
    b1 = jnp.zeros((1, R), jnp.float32)
    for r in range(R):
        rhit = rgrid == r
        for m in range(3):
            W0 = W0 + jnp.where(rhit & (cgrid == 6 * r + m),
                                weights_ref[2 * r, m], 0.0)
            W1 = W1 + jnp.where(rhit & (cgrid == 6 * r + 3 + m),
                                weights_ref[2 * r + 1, m], 0.0)
        rhit1 = rgrid1 == r
        b0 = b0 + jnp.where(rhit1, biases_ref[0, 2 * r], 0.0)
        b1 = b1 + jnp.where(rhit1, biases_ref[0, 2 * r + 1], 0.0)

    s0 = (jnp.dot(scores, W0, preferred_element_type=jnp.float32,
                  precision=_HIGH) + b0).reshape(N, N, R)
    s1 = (jnp.dot(scores, W1, preferred_element_type=jnp.float32,
                  precision=_HIGH) + b1).reshape(N, N, R)
    out_ref[0] = jnp.where(tmask_ref[0] == 0, s0, s1)


def kernel(transitions, type_mask, rules, weights, biases, t_sections, c_sections):
    B, N, _, P = transitions.shape
    R = type_mask.shape[-1]
    Nc = rules.shape[0]
    return pl.pallas_call(
        _tc_body,
        grid=(B,),
        in_specs=[
            pl.BlockSpec((1, N, N, P), lambda b: (b, 0, 0, 0)),
            pl.BlockSpec((1, N, N, R), lambda b: (b, 0, 0, 0)),
            pl.BlockSpec((Nc, 2), lambda b: (0, 0)),
            pl.BlockSpec((16, 3), lambda b: (0, 0)),
            pl.BlockSpec((1, 16), lambda b: (0, 0)),
        ],
        out_specs=pl.BlockSpec((1, N, N, R), lambda b: (b, 0, 0, 0)),
        out_shape=jax.ShapeDtypeStruct((B, N, N, R), transitions.dtype),
        compiler_params=pltpu.CompilerParams(
            dimension_semantics=("arbitrary",)),
    )(transitions, type_mask, rules, weights[:, :, 0], biases.reshape(1, 16))
